# Initial kernel scaffold; baseline (speedup 1.0000x reference)
import jax
import jax.numpy as jnp
from jax.experimental import pallas as pl


def kernel(canoncolor_out, gt_color, pt_offset, mask_pts):
    raise NotImplementedError("write your pallas kernel here")


# speedup vs baseline: 10.5023x; 10.5023x over previous
import functools

import jax
import jax.numpy as jnp
from jax.experimental import pallas as pl
from jax.experimental.pallas import tpu as pltpu

N_PTS = 8192
TOTAL = 65536
WIN = N_PTS + 256
IC = 512
N_STEPS = N_PTS // IC
LC = 1024
N_LC = N_PTS // LC
N_MASKS = 8
N_OBJ = 4
BIG = 1e30


def _body(offs_ref, predT_ref, gtT_ref, maskf_ref, out_ref,
          xt_ref, seg_ref, rowmin_ref, colmin_ref, acc_ref):
    obj = pl.program_id(0)
    s = pl.program_id(1)
    start = offs_ref[obj]
    end = offs_ref[N_OBJ + obj]

    @pl.when(s == 0)
    def _setup():
        base = jnp.minimum((start // 128) * 128, TOTAL - WIN)
        base = pl.multiple_of(base, 128)
        rot = start - base
        for c in range(3):
            wp = predT_ref[c:c + 1, pl.ds(base, WIN)]
            seg_ref[c:c + 1, :] = pltpu.roll(wp, -rot, 1)[:, :N_PTS]
            wg = gtT_ref[c:c + 1, pl.ds(base, WIN)]
            seg_ref[4 + c:5 + c, :] = pltpu.roll(wg, -rot, 1)[:, :N_PTS]
        seg_ref[8:16, :] = (1.0 - maskf_ref[0]) * BIG
        xt_ref[:, :] = jnp.transpose(seg_ref[:, :], (1, 0))
        rowmin_ref[:, :] = jnp.full((N_MASKS, N_PTS), BIG, jnp.float32)
        colmin_ref[:, :] = jnp.full((N_MASKS, N_PTS), BIG, jnp.float32)

    @pl.when((obj == 0) & (s == 0))
    def _init():
        acc_ref[0] = 0.0
        acc_ref[1] = 0.0

    i0 = pl.multiple_of(s * IC, IC)

    def _lane_chunk(lc, _):
        l0 = pl.multiple_of(lc * LC, LC)
        d2a = jnp.zeros((IC, LC), jnp.float32)
        d2b = jnp.zeros((IC, LC), jnp.float32)
        for c in range(3):
            xa = xt_ref[pl.ds(i0, IC), c:c + 1]
            ya = seg_ref[4 + c:5 + c, pl.ds(l0, LC)]
            da = xa - ya
            d2a = d2a + da * da
            xb = xt_ref[pl.ds(i0, IC), 4 + c:5 + c]
            yb = seg_ref[c:c + 1, pl.ds(l0, LC)]
            db = xb - yb
            d2b = d2b + db * db
        for k in range(N_MASKS):
            bigk = xt_ref[pl.ds(i0, IC), 8 + k:9 + k]
            cm = jnp.min(d2a + bigk, axis=0, keepdims=True)
            colmin_ref[k:k + 1, pl.ds(l0, LC)] = jnp.minimum(
                colmin_ref[k:k + 1, pl.ds(l0, LC)], cm)
            rm = jnp.min(d2b + bigk, axis=0, keepdims=True)
            rowmin_ref[k:k + 1, pl.ds(l0, LC)] = jnp.minimum(
                rowmin_ref[k:k + 1, pl.ds(l0, LC)], rm)
        return 0

    jax.lax.fori_loop(0, N_LC, _lane_chunk, 0)

    @pl.when(s == N_STEPS - 1)
    def _finalize():
        part_sum = jnp.float32(0.0)
        num_parts = jnp.float32(0.0)
        for k in range(N_MASKS):
            mrow = maskf_ref[0, k:k + 1, :]
            n = jnp.sum(mrow)
            valid = mrow > 0.0
            rsum = jnp.sum(jnp.where(
                valid, jnp.sqrt(jnp.maximum(rowmin_ref[k:k + 1, :], 0.0)),
                0.0))
            csum = jnp.sum(jnp.where(
                valid, jnp.sqrt(jnp.maximum(colmin_ref[k:k + 1, :], 0.0)),
                0.0))
            loss_k = (rsum + csum) / (2.0 * jnp.maximum(n, 1.0))
            pv = n >= 2.0
            part_sum = part_sum + jnp.where(pv, loss_k, 0.0)
            num_parts = num_parts + pv.astype(jnp.float32)
        obj_valid = (end - start) != 0
        use = obj_valid & (num_parts > 0.0)
        contrib = part_sum / jnp.maximum(num_parts, 1.0)
        acc_ref[0] = acc_ref[0] + jnp.where(use, contrib, 0.0)
        acc_ref[1] = acc_ref[1] + jnp.where(use, 1.0, 0.0)

        @pl.when(obj == N_OBJ - 1)
        def _emit():
            cnt = acc_ref[1]
            val = jnp.where(cnt == 0.0, 0.0,
                            acc_ref[0] / jnp.maximum(cnt, 1.0))
            out_ref[:, :] = jnp.full((1, 1), val, jnp.float32)


@functools.partial(jax.jit, static_argnames=("interpret",))
def _run(predT, gtT, maskf, offs, interpret=False):
    grid_spec = pltpu.PrefetchScalarGridSpec(
        num_scalar_prefetch=1,
        grid=(N_OBJ, N_STEPS),
        in_specs=[
            pl.BlockSpec(predT.shape, lambda o, s, offs: (0, 0)),
            pl.BlockSpec(gtT.shape, lambda o, s, offs: (0, 0)),
            pl.BlockSpec((1, N_MASKS, N_PTS), lambda o, s, offs: (o, 0, 0)),
        ],
        out_specs=pl.BlockSpec((1, 1), lambda o, s, offs: (0, 0)),
        scratch_shapes=[
            pltpu.VMEM((N_PTS, 16), jnp.float32),
            pltpu.VMEM((16, N_PTS), jnp.float32),
            pltpu.VMEM((N_MASKS, N_PTS), jnp.float32),
            pltpu.VMEM((N_MASKS, N_PTS), jnp.float32),
            pltpu.SMEM((2,), jnp.float32),
        ],
    )
    return pl.pallas_call(
        _body,
        grid_spec=grid_spec,
        out_shape=jax.ShapeDtypeStruct((1, 1), jnp.float32),
        compiler_params=pltpu.CompilerParams(
            dimension_semantics=("arbitrary", "arbitrary")),
        interpret=interpret,
    )(offs, predT, gtT, maskf)


def kernel(canoncolor_out, gt_color, pt_offset, mask_pts):
    predT = canoncolor_out.T
    gtT = gt_color.T
    maskf = mask_pts.astype(jnp.float32)
    starts = jnp.concatenate(
        [jnp.zeros((1,), pt_offset.dtype), pt_offset[:N_OBJ - 1]])
    offs = jnp.concatenate([starts, pt_offset[:N_OBJ]]).astype(jnp.int32)
    out = _run(predT, gtT, maskf, offs)
    return out[0, 0]


# speedup vs baseline: 11.7782x; 1.1215x over previous
import functools

import jax
import jax.numpy as jnp
from jax.experimental import pallas as pl
from jax.experimental.pallas import tpu as pltpu

N_PTS = 8192
TOTAL = 65536
WIN = N_PTS + 256
IC = 512
N_STEPS = N_PTS // IC
LC = 1024
N_LC = N_PTS // LC
N_MASKS = 8
N_OBJ = 4
BIG = 1e30


def _body(offs_ref, predT_ref, gtT_ref, maskf_ref, out_ref,
          xt_ref, seg_ref, rowmin_ref, colmin_ref, acc_ref):
    obj = pl.program_id(0)
    s = pl.program_id(1)
    start = offs_ref[obj]
    end = offs_ref[N_OBJ + obj]

    @pl.when(s == 0)
    def _setup():
        base = jnp.minimum((start // 128) * 128, TOTAL - WIN)
        base = pl.multiple_of(base, 128)
        rot = start - base
        for c in range(3):
            wp = predT_ref[c:c + 1, pl.ds(base, WIN)]
            seg_ref[c:c + 1, :] = pltpu.roll(wp, -rot, 1)[:, :N_PTS]
            wg = gtT_ref[c:c + 1, pl.ds(base, WIN)]
            seg_ref[4 + c:5 + c, :] = pltpu.roll(wg, -rot, 1)[:, :N_PTS]
        seg_ref[8:16, :] = (1.0 - maskf_ref[0]) * BIG
        for c in range(3):
            seg_ref[16 + c:17 + c, :] = seg_ref[c:c + 1, :] * -2.0
            seg_ref[20 + c:21 + c, :] = seg_ref[4 + c:5 + c, :] * -2.0
        pp = (seg_ref[0:1, :] * seg_ref[0:1, :]
              + seg_ref[1:2, :] * seg_ref[1:2, :]
              + seg_ref[2:3, :] * seg_ref[2:3, :])
        gg = (seg_ref[4:5, :] * seg_ref[4:5, :]
              + seg_ref[5:6, :] * seg_ref[5:6, :]
              + seg_ref[6:7, :] * seg_ref[6:7, :])
        seg_ref[24:25, :] = pp
        seg_ref[25:26, :] = gg
        seg_ref[3:4, :] = gg
        seg_ref[7:8, :] = pp
        xt_ref[:, :] = jnp.transpose(seg_ref[:, :], (1, 0))
        rowmin_ref[:, :] = jnp.full((N_MASKS, N_PTS), BIG, jnp.float32)
        colmin_ref[:, :] = jnp.full((N_MASKS, N_PTS), BIG, jnp.float32)

    @pl.when((obj == 0) & (s == 0))
    def _init():
        acc_ref[0] = 0.0
        acc_ref[1] = 0.0

    i0 = pl.multiple_of(s * IC, IC)

    def _lane_chunk(lc, _):
        l0 = pl.multiple_of(lc * LC, LC)
        dn = (((1,), (0,)), ((), ()))
        mma = jax.lax.dot_general(
            xt_ref[pl.ds(i0, IC), 16:19], seg_ref[4:7, pl.ds(l0, LC)],
            dn, preferred_element_type=jnp.float32)
        d2a = (xt_ref[pl.ds(i0, IC), 24:25]
               + seg_ref[3:4, pl.ds(l0, LC)] + mma)
        mmb = jax.lax.dot_general(
            xt_ref[pl.ds(i0, IC), 20:23], seg_ref[0:3, pl.ds(l0, LC)],
            dn, preferred_element_type=jnp.float32)
        d2b = (xt_ref[pl.ds(i0, IC), 25:26]
               + seg_ref[7:8, pl.ds(l0, LC)] + mmb)
        for k in range(N_MASKS):
            bigk = xt_ref[pl.ds(i0, IC), 8 + k:9 + k]
            cm = jnp.min(d2a + bigk, axis=0, keepdims=True)
            colmin_ref[k:k + 1, pl.ds(l0, LC)] = jnp.minimum(
                colmin_ref[k:k + 1, pl.ds(l0, LC)], cm)
            rm = jnp.min(d2b + bigk, axis=0, keepdims=True)
            rowmin_ref[k:k + 1, pl.ds(l0, LC)] = jnp.minimum(
                rowmin_ref[k:k + 1, pl.ds(l0, LC)], rm)
        return 0

    jax.lax.fori_loop(0, N_LC, _lane_chunk, 0)

    @pl.when(s == N_STEPS - 1)
    def _finalize():
        part_sum = jnp.float32(0.0)
        num_parts = jnp.float32(0.0)
        for k in range(N_MASKS):
            mrow = maskf_ref[0, k:k + 1, :]
            n = jnp.sum(mrow)
            valid = mrow > 0.0
            rsum = jnp.sum(jnp.where(
                valid, jnp.sqrt(jnp.maximum(rowmin_ref[k:k + 1, :], 0.0)),
                0.0))
            csum = jnp.sum(jnp.where(
                valid, jnp.sqrt(jnp.maximum(colmin_ref[k:k + 1, :], 0.0)),
                0.0))
            loss_k = (rsum + csum) / (2.0 * jnp.maximum(n, 1.0))
            pv = n >= 2.0
            part_sum = part_sum + jnp.where(pv, loss_k, 0.0)
            num_parts = num_parts + pv.astype(jnp.float32)
        obj_valid = (end - start) != 0
        use = obj_valid & (num_parts > 0.0)
        contrib = part_sum / jnp.maximum(num_parts, 1.0)
        acc_ref[0] = acc_ref[0] + jnp.where(use, contrib, 0.0)
        acc_ref[1] = acc_ref[1] + jnp.where(use, 1.0, 0.0)

        @pl.when(obj == N_OBJ - 1)
        def _emit():
            cnt = acc_ref[1]
            val = jnp.where(cnt == 0.0, 0.0,
                            acc_ref[0] / jnp.maximum(cnt, 1.0))
            out_ref[:, :] = jnp.full((1, 1), val, jnp.float32)


@functools.partial(jax.jit, static_argnames=("interpret",))
def _run(predT, gtT, maskf, offs, interpret=False):
    grid_spec = pltpu.PrefetchScalarGridSpec(
        num_scalar_prefetch=1,
        grid=(N_OBJ, N_STEPS),
        in_specs=[
            pl.BlockSpec(predT.shape, lambda o, s, offs: (0, 0)),
            pl.BlockSpec(gtT.shape, lambda o, s, offs: (0, 0)),
            pl.BlockSpec((1, N_MASKS, N_PTS), lambda o, s, offs: (o, 0, 0)),
        ],
        out_specs=pl.BlockSpec((1, 1), lambda o, s, offs: (0, 0)),
        scratch_shapes=[
            pltpu.VMEM((N_PTS, 32), jnp.float32),
            pltpu.VMEM((32, N_PTS), jnp.float32),
            pltpu.VMEM((N_MASKS, N_PTS), jnp.float32),
            pltpu.VMEM((N_MASKS, N_PTS), jnp.float32),
            pltpu.SMEM((2,), jnp.float32),
        ],
    )
    return pl.pallas_call(
        _body,
        grid_spec=grid_spec,
        out_shape=jax.ShapeDtypeStruct((1, 1), jnp.float32),
        compiler_params=pltpu.CompilerParams(
            dimension_semantics=("arbitrary", "arbitrary")),
        interpret=interpret,
    )(offs, predT, gtT, maskf)


def kernel(canoncolor_out, gt_color, pt_offset, mask_pts):
    predT = canoncolor_out.T
    gtT = gt_color.T
    maskf = mask_pts.astype(jnp.float32)
    starts = jnp.concatenate(
        [jnp.zeros((1,), pt_offset.dtype), pt_offset[:N_OBJ - 1]])
    offs = jnp.concatenate([starts, pt_offset[:N_OBJ]]).astype(jnp.int32)
    out = _run(predT, gtT, maskf, offs)
    return out[0, 0]


# speedup vs baseline: 18.3923x; 1.5615x over previous
import functools

import jax
import jax.numpy as jnp
from jax.experimental import pallas as pl
from jax.experimental.pallas import tpu as pltpu

N_PTS = 8192
TOTAL = 65536
WIN = N_PTS + 256
IC = 512
N_STEPS = N_PTS // IC
LC = 1024
N_LC = N_PTS // LC
N_MASKS = 8
N_OBJ = 4
BIG = 1e30


def _body(offs_ref, predT_ref, gtT_ref, maskf_ref, out_ref,
          xt_ref, seg_ref, rowmin_ref, colmin_ref, acc_ref):
    obj = pl.program_id(0)
    s = pl.program_id(1)
    start = offs_ref[obj]
    end = offs_ref[N_OBJ + obj]

    @pl.when(s == 0)
    def _setup():
        base = jnp.minimum((start // 128) * 128, TOTAL - WIN)
        base = pl.multiple_of(base, 128)
        rot = start - base
        for c in range(3):
            wp = predT_ref[c:c + 1, pl.ds(base, WIN)]
            seg_ref[c:c + 1, :] = pltpu.roll(wp, -rot, 1)[:, :N_PTS]
            wg = gtT_ref[c:c + 1, pl.ds(base, WIN)]
            seg_ref[4 + c:5 + c, :] = pltpu.roll(wg, -rot, 1)[:, :N_PTS]
        seg_ref[8:16, :] = (1.0 - maskf_ref[0]) * BIG
        for c in range(3):
            seg_ref[16 + c:17 + c, :] = seg_ref[c:c + 1, :] * -2.0
            seg_ref[20 + c:21 + c, :] = seg_ref[4 + c:5 + c, :] * -2.0
        pp = (seg_ref[0:1, :] * seg_ref[0:1, :]
              + seg_ref[1:2, :] * seg_ref[1:2, :]
              + seg_ref[2:3, :] * seg_ref[2:3, :])
        gg = (seg_ref[4:5, :] * seg_ref[4:5, :]
              + seg_ref[5:6, :] * seg_ref[5:6, :]
              + seg_ref[6:7, :] * seg_ref[6:7, :])
        seg_ref[24:25, :] = pp
        seg_ref[25:26, :] = gg
        seg_ref[3:4, :] = gg
        seg_ref[7:8, :] = pp
        xt_ref[:, :] = jnp.transpose(seg_ref[:, :], (1, 0))
        rowmin_ref[:, :] = jnp.full((N_MASKS, N_PTS), BIG, jnp.bfloat16)
        colmin_ref[:, :] = jnp.full((N_MASKS, N_PTS), BIG, jnp.bfloat16)

    @pl.when((obj == 0) & (s == 0))
    def _init():
        acc_ref[0] = 0.0
        acc_ref[1] = 0.0

    i0 = pl.multiple_of(s * IC, IC)

    def _lane_chunk(lc, _):
        l0 = pl.multiple_of(lc * LC, LC)
        dn = (((1,), (0,)), ((), ()))
        mma = jax.lax.dot_general(
            xt_ref[pl.ds(i0, IC), 16:19], seg_ref[4:7, pl.ds(l0, LC)],
            dn, preferred_element_type=jnp.float32)
        d2a = (xt_ref[pl.ds(i0, IC), 24:25]
               + seg_ref[3:4, pl.ds(l0, LC)] + mma)
        mmb = jax.lax.dot_general(
            xt_ref[pl.ds(i0, IC), 20:23], seg_ref[0:3, pl.ds(l0, LC)],
            dn, preferred_element_type=jnp.float32)
        d2b = (xt_ref[pl.ds(i0, IC), 25:26]
               + seg_ref[7:8, pl.ds(l0, LC)] + mmb)
        d2a16 = d2a.astype(jnp.bfloat16)
        d2b16 = d2b.astype(jnp.bfloat16)
        for k in range(N_MASKS):
            bigk = xt_ref[pl.ds(i0, IC), 8 + k:9 + k].astype(jnp.bfloat16)
            cm = jnp.min(d2a16 + bigk, axis=0, keepdims=True)
            colmin_ref[k:k + 1, pl.ds(l0, LC)] = jnp.minimum(
                colmin_ref[k:k + 1, pl.ds(l0, LC)], cm)
            rm = jnp.min(d2b16 + bigk, axis=0, keepdims=True)
            rowmin_ref[k:k + 1, pl.ds(l0, LC)] = jnp.minimum(
                rowmin_ref[k:k + 1, pl.ds(l0, LC)], rm)
        return 0

    jax.lax.fori_loop(0, N_LC, _lane_chunk, 0)

    @pl.when(s == N_STEPS - 1)
    def _finalize():
        part_sum = jnp.float32(0.0)
        num_parts = jnp.float32(0.0)
        for k in range(N_MASKS):
            mrow = maskf_ref[0, k:k + 1, :]
            n = jnp.sum(mrow)
            valid = mrow > 0.0
            rsum = jnp.sum(jnp.where(
                valid,
                jnp.sqrt(jnp.maximum(
                    rowmin_ref[k:k + 1, :].astype(jnp.float32), 0.0)),
                0.0))
            csum = jnp.sum(jnp.where(
                valid,
                jnp.sqrt(jnp.maximum(
                    colmin_ref[k:k + 1, :].astype(jnp.float32), 0.0)),
                0.0))
            loss_k = (rsum + csum) / (2.0 * jnp.maximum(n, 1.0))
            pv = n >= 2.0
            part_sum = part_sum + jnp.where(pv, loss_k, 0.0)
            num_parts = num_parts + pv.astype(jnp.float32)
        obj_valid = (end - start) != 0
        use = obj_valid & (num_parts > 0.0)
        contrib = part_sum / jnp.maximum(num_parts, 1.0)
        acc_ref[0] = acc_ref[0] + jnp.where(use, contrib, 0.0)
        acc_ref[1] = acc_ref[1] + jnp.where(use, 1.0, 0.0)

        @pl.when(obj == N_OBJ - 1)
        def _emit():
            cnt = acc_ref[1]
            val = jnp.where(cnt == 0.0, 0.0,
                            acc_ref[0] / jnp.maximum(cnt, 1.0))
            out_ref[:, :] = jnp.full((1, 1), val, jnp.float32)


@functools.partial(jax.jit, static_argnames=("interpret",))
def _run(predT, gtT, maskf, offs, interpret=False):
    grid_spec = pltpu.PrefetchScalarGridSpec(
        num_scalar_prefetch=1,
        grid=(N_OBJ, N_STEPS),
        in_specs=[
            pl.BlockSpec(predT.shape, lambda o, s, offs: (0, 0)),
            pl.BlockSpec(gtT.shape, lambda o, s, offs: (0, 0)),
            pl.BlockSpec((1, N_MASKS, N_PTS), lambda o, s, offs: (o, 0, 0)),
        ],
        out_specs=pl.BlockSpec((1, 1), lambda o, s, offs: (0, 0)),
        scratch_shapes=[
            pltpu.VMEM((N_PTS, 32), jnp.float32),
            pltpu.VMEM((32, N_PTS), jnp.float32),
            pltpu.VMEM((N_MASKS, N_PTS), jnp.bfloat16),
            pltpu.VMEM((N_MASKS, N_PTS), jnp.bfloat16),
            pltpu.SMEM((2,), jnp.float32),
        ],
    )
    return pl.pallas_call(
        _body,
        grid_spec=grid_spec,
        out_shape=jax.ShapeDtypeStruct((1, 1), jnp.float32),
        compiler_params=pltpu.CompilerParams(
            dimension_semantics=("arbitrary", "arbitrary")),
        interpret=interpret,
    )(offs, predT, gtT, maskf)


def kernel(canoncolor_out, gt_color, pt_offset, mask_pts):
    predT = canoncolor_out.T
    gtT = gt_color.T
    maskf = mask_pts.astype(jnp.float32)
    starts = jnp.concatenate(
        [jnp.zeros((1,), pt_offset.dtype), pt_offset[:N_OBJ - 1]])
    offs = jnp.concatenate([starts, pt_offset[:N_OBJ]]).astype(jnp.int32)
    out = _run(predT, gtT, maskf, offs)
    return out[0, 0]
